# Initial kernel scaffold; baseline (speedup 1.0000x reference)
import jax
import jax.numpy as jnp
from jax.experimental import pallas as pl


def kernel(x, table):
    raise NotImplementedError("write your pallas kernel here")


# speedup vs baseline: 6.3553x; 6.3553x over previous
import functools

import jax
import jax.numpy as jnp
from jax import lax
from jax.experimental import pallas as pl
from jax.experimental.pallas import tpu as pltpu
from jax.experimental.pallas import tpu_sc as plsc

_INFO = plsc.get_sparse_core_info()
_NC = _INFO.num_cores
_NS = _INFO.num_subcores
_NW = _NC * _NS
_CH = 128


@functools.lru_cache(maxsize=None)
def _build(n_rows: int, d: int):
  assert n_rows % (_NW * _CH) == 0
  chunks_per_w = n_rows // (_NW * _CH)
  rows_per_w = chunks_per_w * _CH

  mesh = plsc.VectorSubcoreMesh(core_axis_name="c", subcore_axis_name="s")

  @functools.partial(
      pl.kernel,
      out_type=jax.ShapeDtypeStruct((n_rows, d), jnp.float32),
      mesh=mesh,
      scratch_types=[
          pltpu.VMEM((chunks_per_w, _CH), jnp.int32),
          pltpu.VMEM((_CH, d), jnp.float32),
          pltpu.SemaphoreType.DMA,
      ],
  )
  def gather_kernel(table_hbm, idx_hbm, out_hbm, idx_v, rows_v, sem):
    wid = lax.axis_index("s") * _NC + lax.axis_index("c")
    base = wid * chunks_per_w
    pltpu.sync_copy(idx_hbm.at[pl.ds(base, chunks_per_w)], idx_v)

    @pl.loop(0, chunks_per_w)
    def _step(j):
      pltpu.async_copy(table_hbm.at[idx_v.at[j]], rows_v, sem).wait()
      pltpu.sync_copy(rows_v, out_hbm.at[pl.ds((base + j) * _CH, _CH)])

  return gather_kernel


def kernel(x, table):
  b, h = x.shape
  v, d = table.shape
  n_rows = b * h
  idx2d = x.reshape(n_rows // _CH, _CH).astype(jnp.int32)
  out = _build(n_rows, d)(table, idx2d)
  return out.reshape(b, h, d)


# speedup vs baseline: 9.1501x; 1.4397x over previous
import functools

import jax
import jax.numpy as jnp
from jax import lax
from jax.experimental import pallas as pl
from jax.experimental.pallas import tpu as pltpu
from jax.experimental.pallas import tpu_sc as plsc

_INFO = plsc.get_sparse_core_info()
_NC = _INFO.num_cores
_NS = _INFO.num_subcores
_NW = _NC * _NS
_CH = 128
_NBUF = 4


@functools.lru_cache(maxsize=None)
def _build(n_rows: int, d: int):
  assert n_rows % (_NW * _CH) == 0
  chunks_per_w = n_rows // (_NW * _CH)
  assert chunks_per_w % _NBUF == 0 and chunks_per_w >= 2 * _NBUF

  mesh = plsc.VectorSubcoreMesh(core_axis_name="c", subcore_axis_name="s")

  @functools.partial(
      pl.kernel,
      out_type=jax.ShapeDtypeStruct((n_rows, d), jnp.float32),
      mesh=mesh,
      scratch_types=[
          pltpu.VMEM((chunks_per_w, _CH), jnp.int32),
          [pltpu.VMEM((_CH, d), jnp.float32)] * _NBUF,
          [pltpu.SemaphoreType.DMA] * _NBUF,
          [pltpu.SemaphoreType.DMA] * _NBUF,
      ],
  )
  def gather_kernel(table_hbm, idx_hbm, out_hbm, idx_v, bufs, gsems, osems):
    wid = lax.axis_index("s") * _NC + lax.axis_index("c")
    base = wid * chunks_per_w
    pltpu.sync_copy(idx_hbm.at[pl.ds(base, chunks_per_w)], idx_v)

    def start_gather(b, j):
      pltpu.async_copy(table_hbm.at[idx_v.at[j]], bufs[b], gsems[b])

    def wait_gather(b, j):
      pltpu.make_async_copy(table_hbm.at[idx_v.at[j]], bufs[b], gsems[b]).wait()

    def out_slice(j):
      return out_hbm.at[pl.ds((base + j) * _CH, _CH)]

    def start_writeback(b, j):
      pltpu.async_copy(bufs[b], out_slice(j), osems[b])

    def wait_writeback(b, j):
      pltpu.make_async_copy(bufs[b], out_slice(j), osems[b]).wait()

    for b in range(_NBUF):
      start_gather(b, b)

    @pl.loop(0, chunks_per_w - _NBUF, step=_NBUF)
    def _step(j0):
      for b in range(_NBUF):
        wait_gather(b, j0 + b)
        start_writeback(b, j0 + b)
      for b in range(_NBUF):
        wait_writeback(b, j0 + b)
        start_gather(b, j0 + _NBUF + b)

    j0 = chunks_per_w - _NBUF
    for b in range(_NBUF):
      wait_gather(b, j0 + b)
      start_writeback(b, j0 + b)
    for b in range(_NBUF):
      wait_writeback(b, j0 + b)

  return gather_kernel


def kernel(x, table):
  b, h = x.shape
  v, d = table.shape
  n_rows = b * h
  idx2d = x.reshape(n_rows // _CH, _CH).astype(jnp.int32)
  out = _build(n_rows, d)(table, idx2d)
  return out.reshape(b, h, d)
